# baseline (device time: 30917 ns/iter reference)
import jax
import jax.numpy as jnp
from jax import lax
from jax.experimental import pallas as pl
from jax.experimental.pallas import tpu as pltpu

N_DEV = 8
BLK = 64

F_H0_CW, F_H0_CCW, F_H1_CW, F_H1_CCW, F_H2, F_CH0, F_CH1 = range(7)


def kernel(x, Wq, K_ext, V_ext, Wo):
    B, Sq_l, D = x.shape
    _, Skv_l, Hq, Dh = K_ext.shape
    HD = Hq * Dh

    k2 = K_ext.reshape(B, Skv_l, HD).astype(jnp.bfloat16)
    v2 = V_ext.reshape(B, Skv_l, HD).astype(jnp.bfloat16)

    def body(x_ref, wq_ref, k_ref, v_ref, wo_ref, out_ref,
             kbuf, vbuf, send_sems, recv_sems):
        my = lax.axis_index("i")

        def r2m(p):
            return jnp.where(p < 4, p, 11 - p)

        def ring(p):
            return r2m(lax.rem(p + N_DEV, N_DEV))

        r = r2m(my)
        right = ring(r + 1)
        left = ring(r - 1)
        odd = lax.rem(r, 2) == 1
        chord = jnp.where(odd, ring(r - 3), ring(r + 3))

        barrier = pltpu.get_barrier_semaphore()
        for nbr in (left, right, chord):
            pl.semaphore_signal(barrier, inc=1, device_id=(nbr,),
                                device_id_type=pl.DeviceIdType.MESH)
        pl.semaphore_wait(barrier, 3)

        def copy(src_ref, dst_ref, t, hb, flow, dst_dev):
            rd = pltpu.make_async_remote_copy(
                src_ref=src_ref, dst_ref=dst_ref,
                send_sem=send_sems.at[t, hb, flow],
                recv_sem=recv_sems.at[t, hb, flow],
                device_id=(dst_dev,), device_id_type=pl.DeviceIdType.MESH)
            rd.start()
            return rd

        def start_t(step, t, hb):
            buf = (kbuf, vbuf)[t]
            if step == 0:
                inp = (k_ref, v_ref)[t]
                return [copy(inp.at[hb], buf.at[my, hb], t, hb, F_H0_CW, right),
                        copy(inp.at[hb], buf.at[my, hb], t, hb, F_H0_CCW, left),
                        copy(inp.at[hb], buf.at[my, hb], t, hb, F_CH0, chord)]
            if step == 1:
                s_ch = jnp.where(odd, ring(r + 1), ring(r - 1))
                return [copy(buf.at[ring(r - 1), hb], buf.at[ring(r - 1), hb],
                             t, hb, F_H1_CW, right),
                        copy(buf.at[ring(r + 1), hb], buf.at[ring(r + 1), hb],
                             t, hb, F_H1_CCW, left),
                        copy(buf.at[s_ch, hb], buf.at[s_ch, hb],
                             t, hb, F_CH1, chord)]
            s2 = jnp.where(odd, ring(r - 2), ring(r + 2))
            return [copy(buf.at[s2, hb], buf.at[s2, hb], t, hb, F_H2,
                         jnp.where(odd, right, left))]

        qblocks_per_shard = Sq_l // BLK
        kblocks_per_shard = Skv_l // BLK

        state = {}
        for b in range(B):
            for hq in range(Hq):
                state[(b, hq)] = (
                    jnp.full((Sq_l, 1), -1e30, jnp.float32),
                    jnp.zeros((Sq_l, 1), jnp.float32),
                    jnp.zeros((Sq_l, Dh), jnp.float32),
                )

        def process_chunk(o, chunk_b, qf):
            for b in range(B):
                kc, vc = chunk_b(b)
                qi = lax.broadcasted_iota(jnp.int32, (Sq_l, Skv_l), 0)
                ki = lax.broadcasted_iota(jnp.int32, (Sq_l, Skv_l), 1)
                qb = qi // BLK + my * qblocks_per_shard
                kb = ki // BLK + o * kblocks_per_shard
                mask = (qb == kb) | (kb == 0) | (lax.rem(qb + kb, 3) == 0)
                for hq in range(Hq):
                    qbh = qf[b * Sq_l:(b + 1) * Sq_l, hq * Dh:(hq + 1) * Dh]
                    s = lax.dot_general(
                        qbh.astype(jnp.bfloat16), kc[:, hq * Dh:(hq + 1) * Dh],
                        (((1,), (1,)), ((), ())),
                        preferred_element_type=jnp.float32)
                    s = jnp.where(mask, s * 0.125, -1e9)
                    m_old, den, ctx = state[(b, hq)]
                    m_new = jnp.maximum(m_old, s.max(axis=1, keepdims=True))
                    scale = jnp.exp(m_old - m_new)
                    e = jnp.exp(s - m_new)
                    den = den * scale + e.sum(axis=1, keepdims=True)
                    ctx = ctx * scale + jnp.dot(
                        e.astype(jnp.bfloat16), vc[:, hq * Dh:(hq + 1) * Dh],
                        preferred_element_type=jnp.float32)
                    state[(b, hq)] = (m_new, den, ctx)

        def buf_chunk(o):
            return lambda b: (kbuf[o, b], vbuf[o, b])

        Q_ORDER = [(0, 0), (0, 1), (1, 0), (1, 1)]
        sent = []
        s0 = {q: start_t(0, *q) for q in Q_ORDER}
        sent += [rd for q in Q_ORDER for rd in s0[q]]
        xf = x_ref[...].reshape(B * Sq_l, D)
        qf = jnp.dot(xf, wq_ref[...], preferred_element_type=jnp.float32)
        process_chunk(my, lambda b: (k_ref[b], v_ref[b]), qf)

        s1 = {}
        for q in Q_ORDER:
            s0[q][0].wait_recv()
            s0[q][1].wait_recv()
            s1[q] = start_t(1, *q)
            sent += s1[q]
        process_chunk(ring(r - 1), buf_chunk(ring(r - 1)), qf)
        process_chunk(ring(r + 1), buf_chunk(ring(r + 1)), qf)
        for q in Q_ORDER:
            s0[q][2].wait_recv()
        process_chunk(chord, buf_chunk(chord), qf)

        s2 = {}
        for q in Q_ORDER:
            s1[q][0].wait_recv()
            s1[q][1].wait_recv()
            s2[q] = start_t(2, *q)
            sent += s2[q]
        process_chunk(ring(r - 2), buf_chunk(ring(r - 2)), qf)
        process_chunk(ring(r + 2), buf_chunk(ring(r + 2)), qf)
        for q in Q_ORDER:
            s1[q][2].wait_recv()
        process_chunk(ring(r + 4), buf_chunk(ring(r + 4)), qf)

        for q in Q_ORDER:
            s2[q][0].wait_recv()
        o_last = jnp.where(odd, ring(r + 3), ring(r - 3))
        process_chunk(o_last, buf_chunk(o_last), qf)

        for b in range(B):
            ctx_heads = []
            for hq in range(Hq):
                _, den, ctx = state[(b, hq)]
                ctx_heads.append(ctx / den)
            ctx_b = jnp.concatenate(ctx_heads, axis=1)
            out_ref[b] = jnp.dot(ctx_b.astype(jnp.bfloat16),
                                 wo_ref[...].astype(jnp.bfloat16),
                                 preferred_element_type=jnp.float32)

        for rd in sent:
            rd.wait_send()

    return pl.pallas_call(
        body,
        out_shape=jax.ShapeDtypeStruct((B, Sq_l, D), jnp.float32),
        in_specs=[pl.BlockSpec(memory_space=pltpu.VMEM)] * 5,
        out_specs=pl.BlockSpec(memory_space=pltpu.VMEM),
        scratch_shapes=[
            pltpu.VMEM((N_DEV, B, Skv_l, HD), jnp.bfloat16),
            pltpu.VMEM((N_DEV, B, Skv_l, HD), jnp.bfloat16),
            pltpu.SemaphoreType.DMA((2, 2, 7)),
            pltpu.SemaphoreType.DMA((2, 2, 7)),
        ],
        compiler_params=pltpu.CompilerParams(collective_id=0),
    )(x, Wq, k2, v2, Wo)


# device time: 30543 ns/iter; 1.0122x vs baseline; 1.0122x over previous
import jax
import jax.numpy as jnp
from jax import lax
from jax.experimental import pallas as pl
from jax.experimental.pallas import tpu as pltpu

N_DEV = 8
BLK = 64

F_H0_CW, F_H0_CCW, F_H1_CW, F_H1_CCW, F_H2, F_CH0, F_CH1 = range(7)


def kernel(x, Wq, K_ext, V_ext, Wo):
    B, Sq_l, D = x.shape
    _, Skv_l, Hq, Dh = K_ext.shape
    HD = Hq * Dh

    k2 = K_ext.reshape(B, Skv_l, HD).astype(jnp.bfloat16)
    v2 = V_ext.reshape(B, Skv_l, HD).astype(jnp.bfloat16)

    def body(x_ref, wq_ref, k_ref, v_ref, wo_ref, out_ref,
             kbuf, vbuf, send_sems, recv_sems):
        my = lax.axis_index("i")

        def r2m(p):
            return jnp.where(p < 4, p, 11 - p)

        def ring(p):
            return r2m(lax.rem(p + N_DEV, N_DEV))

        r = r2m(my)
        right = ring(r + 1)
        left = ring(r - 1)
        odd = lax.rem(r, 2) == 1
        chord = jnp.where(odd, ring(r - 3), ring(r + 3))

        barrier = pltpu.get_barrier_semaphore()
        for nbr in (left, right, chord):
            pl.semaphore_signal(barrier, inc=1, device_id=(nbr,),
                                device_id_type=pl.DeviceIdType.MESH)
        pl.semaphore_wait(barrier, 3)

        def copy(src_ref, dst_ref, t, flow, dst_dev):
            rd = pltpu.make_async_remote_copy(
                src_ref=src_ref, dst_ref=dst_ref,
                send_sem=send_sems.at[t, flow],
                recv_sem=recv_sems.at[t, flow],
                device_id=(dst_dev,), device_id_type=pl.DeviceIdType.MESH)
            rd.start()
            return rd

        def start_t(step, t):
            buf = (kbuf, vbuf)[t]
            if step == 0:
                inp = (k_ref, v_ref)[t]
                return [copy(inp, buf.at[my], t, F_H0_CW, right),
                        copy(inp, buf.at[my], t, F_H0_CCW, left),
                        copy(inp, buf.at[my], t, F_CH0, chord)]
            if step == 1:
                s_ch = jnp.where(odd, ring(r + 1), ring(r - 1))
                return [copy(buf.at[ring(r - 1)], buf.at[ring(r - 1)],
                             t, F_H1_CW, right),
                        copy(buf.at[ring(r + 1)], buf.at[ring(r + 1)],
                             t, F_H1_CCW, left),
                        copy(buf.at[s_ch], buf.at[s_ch], t, F_CH1, chord)]
            s2 = jnp.where(odd, ring(r - 2), ring(r + 2))
            return [copy(buf.at[s2], buf.at[s2], t, F_H2,
                         jnp.where(odd, right, left))]

        qblocks_per_shard = Sq_l // BLK
        kblocks_per_shard = Skv_l // BLK

        state = {}
        for b in range(B):
            for hq in range(Hq):
                state[(b, hq)] = (
                    jnp.full((Sq_l, 1), -1e30, jnp.float32),
                    jnp.zeros((Sq_l, 1), jnp.float32),
                    jnp.zeros((Sq_l, Dh), jnp.float32),
                )

        def process_chunk(o, chunk_b, qf):
            for b in range(B):
                kc, vc = chunk_b(b)
                qi = lax.broadcasted_iota(jnp.int32, (Sq_l, Skv_l), 0)
                ki = lax.broadcasted_iota(jnp.int32, (Sq_l, Skv_l), 1)
                qb = qi // BLK + my * qblocks_per_shard
                kb = ki // BLK + o * kblocks_per_shard
                mask = (qb == kb) | (kb == 0) | (lax.rem(qb + kb, 3) == 0)
                for hq in range(Hq):
                    qbh = qf[b * Sq_l:(b + 1) * Sq_l, hq * Dh:(hq + 1) * Dh]
                    s = lax.dot_general(
                        qbh.astype(jnp.bfloat16), kc[:, hq * Dh:(hq + 1) * Dh],
                        (((1,), (1,)), ((), ())),
                        preferred_element_type=jnp.float32)
                    s = jnp.where(mask, s * 0.125, -1e9)
                    m_old, den, ctx = state[(b, hq)]
                    m_new = jnp.maximum(m_old, s.max(axis=1, keepdims=True))
                    scale = jnp.exp(m_old - m_new)
                    e = jnp.exp(s - m_new)
                    den = den * scale + e.sum(axis=1, keepdims=True)
                    ctx = ctx * scale + jnp.dot(
                        e.astype(jnp.bfloat16), vc[:, hq * Dh:(hq + 1) * Dh],
                        preferred_element_type=jnp.float32)
                    state[(b, hq)] = (m_new, den, ctx)

        def buf_chunk(o):
            return lambda b: (kbuf[o, b], vbuf[o, b])

        sent = []
        k_rds = start_t(0, 0)
        v_rds = start_t(0, 1)
        sent += k_rds + v_rds
        xf = x_ref[...].reshape(B * Sq_l, D)
        qf = jnp.dot(xf, wq_ref[...], preferred_element_type=jnp.float32)
        process_chunk(my, lambda b: (k_ref[b], v_ref[b]), qf)

        k_rds[0].wait_recv()
        k_rds[1].wait_recv()
        k1 = start_t(1, 0)
        v_rds[0].wait_recv()
        v_rds[1].wait_recv()
        v1 = start_t(1, 1)
        sent += k1 + v1
        process_chunk(ring(r - 1), buf_chunk(ring(r - 1)), qf)
        process_chunk(ring(r + 1), buf_chunk(ring(r + 1)), qf)
        k_rds[2].wait_recv()
        v_rds[2].wait_recv()
        process_chunk(chord, buf_chunk(chord), qf)

        k1[0].wait_recv()
        k1[1].wait_recv()
        k2_rd = start_t(2, 0)
        v1[0].wait_recv()
        v1[1].wait_recv()
        v2_rd = start_t(2, 1)
        sent += k2_rd + v2_rd
        process_chunk(ring(r - 2), buf_chunk(ring(r - 2)), qf)
        process_chunk(ring(r + 2), buf_chunk(ring(r + 2)), qf)
        k1[2].wait_recv()
        v1[2].wait_recv()
        process_chunk(ring(r + 4), buf_chunk(ring(r + 4)), qf)

        k2_rd[0].wait_recv()
        v2_rd[0].wait_recv()
        o_last = jnp.where(odd, ring(r + 3), ring(r - 3))
        process_chunk(o_last, buf_chunk(o_last), qf)

        for b in range(B):
            ctx_heads = []
            for hq in range(Hq):
                _, den, ctx = state[(b, hq)]
                ctx_heads.append(ctx / den)
            ctx_b = jnp.concatenate(ctx_heads, axis=1)
            out_ref[b] = jnp.dot(ctx_b.astype(jnp.bfloat16),
                                 wo_ref[...].astype(jnp.bfloat16),
                                 preferred_element_type=jnp.float32)

        for rd in sent:
            rd.wait_send()

    return pl.pallas_call(
        body,
        out_shape=jax.ShapeDtypeStruct((B, Sq_l, D), jnp.float32),
        in_specs=[pl.BlockSpec(memory_space=pltpu.VMEM)] * 5,
        out_specs=pl.BlockSpec(memory_space=pltpu.VMEM),
        scratch_shapes=[
            pltpu.VMEM((N_DEV, B, Skv_l, HD), jnp.bfloat16),
            pltpu.VMEM((N_DEV, B, Skv_l, HD), jnp.bfloat16),
            pltpu.SemaphoreType.DMA((2, 7)),
            pltpu.SemaphoreType.DMA((2, 7)),
        ],
        compiler_params=pltpu.CompilerParams(collective_id=0),
    )(x, Wq, k2, v2, Wo)
